# Initial kernel scaffold; baseline (speedup 1.0000x reference)
import jax
import jax.numpy as jnp
from jax.experimental import pallas as pl


def kernel(x):
    raise NotImplementedError("write your pallas kernel here")


# speedup vs baseline: 8.6569x; 8.6569x over previous
import jax
import jax.numpy as jnp
from jax.experimental import pallas as pl

_N = 8192
_BR = 256


def _body(x_ref, o_ref):
    i = pl.program_id(0)
    o_ref[...] = x_ref[...]
    c0 = i * _BR
    sub = x_ref[:, pl.ds(c0, _BR)]
    rows = jax.lax.broadcasted_iota(jnp.int32, (_BR, _BR), 0)
    cols = jax.lax.broadcasted_iota(jnp.int32, (_BR, _BR), 1)
    o_ref[:, pl.ds(c0, _BR)] = jnp.where(rows == cols, jnp.tanh(sub), sub)


def kernel(x):
    n = x.shape[0]
    return pl.pallas_call(
        _body,
        grid=(n // _BR,),
        in_specs=[pl.BlockSpec((_BR, n), lambda i: (i, 0))],
        out_specs=pl.BlockSpec((_BR, n), lambda i: (i, 0)),
        out_shape=jax.ShapeDtypeStruct((n, n), x.dtype),
    )(x)
